# baseline (speedup 1.0000x reference)
import functools

import jax
import jax.numpy as jnp
from jax import lax
from jax.experimental import pallas as pl
from jax.experimental.pallas import tpu as pltpu
from jax.experimental.pallas import tpu_sc as plsc

N = 10000
D = 128
ROWS = 10240
NC = 2
NS = 16
NW = NC * NS
CHUNK = 128
RPT = ROWS // NS
ZB = 64
ZB2 = 32
DEGW = 128
G = 8
FAST_C = 0
FRAC = 0.65

_mesh = plsc.VectorSubcoreMesh(core_axis_name="c", subcore_axis_name="s")


def _split(totc):
  cpw_f = int(round(totc * FRAC / NS / G)) * G
  cpw_s = totc // NS - cpw_f
  assert cpw_s > 0 and cpw_s % G == 0
  return cpw_f, cpw_s


def _init_const_buf(ref, rows, width, value):
  per_row = width // 16

  def body(t, _):
    i = t // per_row
    k = t % per_row
    ref[i, pl.ds(k * 16, 16)] = jnp.full((16,), value, jnp.float32)
    return 0

  lax.fori_loop(0, rows * per_row, body, 0)


def _make_deg_kernel(totc):
  cpw = totc // NW

  @functools.partial(
      pl.kernel,
      out_type=jax.ShapeDtypeStruct((NC, ROWS, DEGW), jnp.float32),
      mesh=_mesh,
      scratch_types=[
          pltpu.VMEM((cpw, CHUNK), jnp.int32),
          pltpu.VMEM((CHUNK, DEGW), jnp.float32),
          pltpu.VMEM((ZB, DEGW), jnp.float32),
          pltpu.VMEM_SHARED((ROWS, DEGW), jnp.float32),
      ],
  )
  def deg_kernel(dst_hbm, out_hbm, idx_v, ones_v, zb_v, acc):
    c = lax.axis_index("c")
    s = lax.axis_index("s")
    wid = c * NS + s
    _init_const_buf(ones_v, CHUNK, DEGW, 1.0)
    _init_const_buf(zb_v, ZB, DEGW, 0.0)

    def zbody(t, _):
      pltpu.sync_copy(zb_v, acc.at[pl.ds(s * RPT + t * ZB, ZB)])
      return 0
    lax.fori_loop(0, RPT // ZB, zbody, 0)
    plsc.subcore_barrier()

    pltpu.sync_copy(dst_hbm.at[pl.ds(wid * cpw, cpw)], idx_v)

    def body(j, _):
      pltpu.sync_copy(ones_v, acc.at[idx_v.at[j]], add=True)
      return 0
    lax.fori_loop(0, cpw, body, 0)
    plsc.subcore_barrier()

    pltpu.sync_copy(acc.at[pl.ds(s * RPT, RPT)],
                    out_hbm.at[c, pl.ds(s * RPT, RPT)])

  return deg_kernel


def _make_scatter_kernel(totc):
  cpw_f, cpw_s = _split(totc)
  nch_f = NS * cpw_f

  @functools.partial(
      pl.kernel,
      out_type=jax.ShapeDtypeStruct((NC, ROWS, D), jnp.float32),
      mesh=_mesh,
      scratch_types=[
          pltpu.VMEM((cpw_f, CHUNK), jnp.int32),
          pltpu.VMEM((cpw_f, CHUNK), jnp.int32),
          pltpu.VMEM((CHUNK, D), jnp.float32),
          pltpu.VMEM((ZB2, D), jnp.float32),
          pltpu.VMEM_SHARED((ROWS, D), jnp.float32),
          pltpu.SemaphoreType.DMA,
      ],
  )
  def scat_kernel(y_hbm, src_hbm, dst_hbm, out_hbm, idx_s, idx_d, rows_v,
                  zb_v, acc, sem):
    c = lax.axis_index("c")
    s = lax.axis_index("s")
    fast = c == FAST_C
    base = jnp.where(fast, s * cpw_f, nch_f + s * cpw_s)
    cnt = jnp.where(fast, cpw_f, cpw_s)
    _init_const_buf(zb_v, ZB2, D, 0.0)

    def zbody(t, _):
      pltpu.sync_copy(zb_v, acc.at[pl.ds(s * RPT + t * ZB2, ZB2)])
      return 0
    lax.fori_loop(0, RPT // ZB2, zbody, 0)
    plsc.subcore_barrier()

    pltpu.sync_copy(src_hbm.at[pl.ds(base, cpw_f)], idx_s)
    pltpu.sync_copy(dst_hbm.at[pl.ds(base, cpw_f)], idx_d)

    def body(j, _):
      pltpu.async_copy(y_hbm.at[idx_s.at[j]], rows_v, sem).wait()
      pltpu.sync_copy(rows_v, acc.at[idx_d.at[j]], add=True)
      return 0

    lax.fori_loop(0, cnt, body, 0)
    plsc.subcore_barrier()

    pltpu.sync_copy(acc.at[pl.ds(s * RPT, RPT)],
                    out_hbm.at[c, pl.ds(s * RPT, RPT)])

  return scat_kernel


def _tc_prep(x_p, W1, degp):
  def body(x_ref, w_ref, dp_ref, y_ref, dinv_ref):
    deg = dp_ref[0, :, 0:1] + dp_ref[1, :, 0:1] + 1.0
    dinv = lax.rsqrt(jnp.maximum(deg, 1e-12))
    xw = jnp.dot(x_ref[...], w_ref[...], preferred_element_type=jnp.float32)
    y_ref[...] = xw * dinv
    dinv_ref[...] = dinv

  return pl.pallas_call(
      body,
      out_shape=(jax.ShapeDtypeStruct((ROWS, D), jnp.float32),
                 jax.ShapeDtypeStruct((ROWS, 1), jnp.float32)),
  )(x_p, W1, degp)


def _tc_mid(accp, y1, dinv, b1, gamma, beta, W2):
  def body(a_ref, y_ref, di_ref, b1_ref, g_ref, be_ref, w2_ref,
           h_ref, y2_ref):
    dinv = di_ref[...]
    h = (a_ref[0] + a_ref[1] + y_ref[...]) * dinv + b1_ref[...]
    h_ref[...] = h
    hv = h[:N]
    mu = jnp.mean(hv, axis=0, keepdims=True)
    var = jnp.mean(jnp.square(hv - mu), axis=0, keepdims=True)
    hn = (h - mu) * lax.rsqrt(var + 1e-5) * g_ref[...] + be_ref[...]
    hr = jnp.maximum(hn, 0.0)
    hw = jnp.dot(hr, w2_ref[...], preferred_element_type=jnp.float32)
    y2_ref[...] = hw * dinv

  return pl.pallas_call(
      body,
      out_shape=(jax.ShapeDtypeStruct((ROWS, D), jnp.float32),
                 jax.ShapeDtypeStruct((ROWS, D), jnp.float32)),
  )(accp, y1, dinv, b1, gamma, beta, W2)


def _tc_final(accp, y2, dinv, b2):
  def body(a_ref, y_ref, di_ref, b2_ref, o_ref):
    o_ref[...] = (a_ref[0] + a_ref[1] + y_ref[...]) * di_ref[...] + b2_ref[...]

  return pl.pallas_call(
      body,
      out_shape=jax.ShapeDtypeStruct((ROWS, D), jnp.float32),
  )(accp, y2, dinv, b2)


@jax.jit
def kernel(x, edge_index, W1, b1, gamma, beta, W2, b2):
  E = edge_index.shape[1]
  totc = -(-E // CHUNK)
  totc = -(-totc // (NW * G)) * (NW * G)
  cpw_f, _ = _split(totc)
  e_pad = (totc + cpw_f) * CHUNK

  src = edge_index[0]
  dst = edge_index[1]
  pad = jnp.full((e_pad - E,), N, jnp.int32)
  src_p = jnp.concatenate([src, pad]).reshape(totc + cpw_f, CHUNK)
  dst_p = jnp.concatenate([dst, pad]).reshape(totc + cpw_f, CHUNK)

  x_p = jnp.pad(x, ((0, ROWS - N), (0, 0)))
  b1r = b1.reshape(1, D)
  b2r = b2.reshape(1, D)
  gr = gamma.reshape(1, D)
  ber = beta.reshape(1, D)

  degp = _make_deg_kernel(totc)(dst_p)
  y1, dinv = _tc_prep(x_p, W1, degp)
  acc1 = _make_scatter_kernel(totc)(y1, src_p, dst_p)
  hidden_p, y2 = _tc_mid(acc1, y1, dinv, b1r, gr, ber, W2)
  acc2 = _make_scatter_kernel(totc)(y2, src_p, dst_p)
  out_p = _tc_final(acc2, y2, dinv, b2r)

  return out_p[:N], hidden_p[:N]


# speedup vs baseline: 1.0942x; 1.0942x over previous
import functools

import jax
import jax.numpy as jnp
from jax import lax
from jax.experimental import pallas as pl
from jax.experimental.pallas import tpu as pltpu
from jax.experimental.pallas import tpu_sc as plsc

N = 10000
D = 128
ROWS = 10240
NC = 2
NS = 16
NW = NC * NS
CHUNK = 128
RPT = ROWS // NS
ZB = 64
ZB2 = 32
DEGW = 128
G = 8
FAST_C = 0
FRAC = 0.65

_mesh = plsc.VectorSubcoreMesh(core_axis_name="c", subcore_axis_name="s")


def _split(totc):
  cpw_f = int(round(totc * FRAC / NS / G)) * G
  cpw_s = totc // NS - cpw_f
  assert cpw_s > 0 and cpw_s % G == 0
  return cpw_f, cpw_s


def _init_const_buf(ref, rows, width, value):
  per_row = width // 16

  def body(t, _):
    i = t // per_row
    k = t % per_row
    ref[i, pl.ds(k * 16, 16)] = jnp.full((16,), value, jnp.float32)
    return 0

  lax.fori_loop(0, rows * per_row, body, 0)


def _make_deg_kernel(totc):
  cpw = totc // NW

  @functools.partial(
      pl.kernel,
      out_type=jax.ShapeDtypeStruct((NC, ROWS, DEGW), jnp.float32),
      mesh=_mesh,
      scratch_types=[
          pltpu.VMEM((cpw, CHUNK), jnp.int32),
          pltpu.VMEM((CHUNK, DEGW), jnp.float32),
          pltpu.VMEM((ZB, DEGW), jnp.float32),
          pltpu.VMEM_SHARED((ROWS, DEGW), jnp.float32),
      ],
  )
  def deg_kernel(dst_hbm, out_hbm, idx_v, ones_v, zb_v, acc):
    c = lax.axis_index("c")
    s = lax.axis_index("s")
    wid = c * NS + s
    _init_const_buf(ones_v, CHUNK, DEGW, 1.0)
    _init_const_buf(zb_v, ZB, DEGW, 0.0)

    def zbody(t, _):
      pltpu.sync_copy(zb_v, acc.at[pl.ds(s * RPT + t * ZB, ZB)])
      return 0
    lax.fori_loop(0, RPT // ZB, zbody, 0)
    plsc.subcore_barrier()

    pltpu.sync_copy(dst_hbm.at[pl.ds(wid * cpw, cpw)], idx_v)

    def body(j, _):
      pltpu.sync_copy(ones_v, acc.at[idx_v.at[j]], add=True)
      return 0
    lax.fori_loop(0, cpw, body, 0)
    plsc.subcore_barrier()

    pltpu.sync_copy(acc.at[pl.ds(s * RPT, RPT)],
                    out_hbm.at[c, pl.ds(s * RPT, RPT)])

  return deg_kernel


def _make_scatter_kernel(totc):
  cpw = totc // NW

  @functools.partial(
      pl.kernel,
      out_type=jax.ShapeDtypeStruct((NC, ROWS, D), jnp.float32),
      mesh=_mesh,
      scratch_types=[
          pltpu.VMEM((cpw, CHUNK), jnp.int32),
          pltpu.VMEM((cpw, CHUNK), jnp.int32),
          pltpu.VMEM((CHUNK, D), jnp.float32),
          pltpu.VMEM((ZB, D), jnp.float32),
          pltpu.VMEM_SHARED((ROWS, D), jnp.float32),
          pltpu.SemaphoreType.DMA,
      ],
  )
  def scat_kernel(y_hbm, src_hbm, dst_hbm, out_hbm, idx_s, idx_d, rows_v,
                  zb_v, acc, sem):
    c = lax.axis_index("c")
    s = lax.axis_index("s")
    wid = c * NS + s
    _init_const_buf(zb_v, ZB, D, 0.0)

    def zbody(t, _):
      pltpu.sync_copy(zb_v, acc.at[pl.ds(s * RPT + t * ZB, ZB)])
      return 0
    lax.fori_loop(0, RPT // ZB, zbody, 0)
    plsc.subcore_barrier()

    pltpu.sync_copy(src_hbm.at[pl.ds(wid * cpw, cpw)], idx_s)
    pltpu.sync_copy(dst_hbm.at[pl.ds(wid * cpw, cpw)], idx_d)

    def body(j, _):
      pltpu.async_copy(y_hbm.at[idx_s.at[j]], rows_v, sem).wait()
      pltpu.sync_copy(rows_v, acc.at[idx_d.at[j]], add=True)
      return 0

    lax.fori_loop(0, cpw, body, 0)
    plsc.subcore_barrier()

    pltpu.sync_copy(acc.at[pl.ds(s * RPT, RPT)],
                    out_hbm.at[c, pl.ds(s * RPT, RPT)])

  return scat_kernel


def _tc_matmul(x_p, W1):
  def body(x_ref, w_ref, o_ref):
    o_ref[...] = jnp.dot(x_ref[...], w_ref[...],
                         preferred_element_type=jnp.float32)

  return pl.pallas_call(
      body,
      out_shape=jax.ShapeDtypeStruct((ROWS, D), jnp.float32),
  )(x_p, W1)


def _tc_scale(xw, degp):
  def body(xw_ref, dp_ref, y_ref, dinv_ref):
    deg = dp_ref[0, :, 0:1] + dp_ref[1, :, 0:1] + 1.0
    dinv = lax.rsqrt(jnp.maximum(deg, 1e-12))
    y_ref[...] = xw_ref[...] * dinv
    dinv_ref[...] = dinv

  return pl.pallas_call(
      body,
      out_shape=(jax.ShapeDtypeStruct((ROWS, D), jnp.float32),
                 jax.ShapeDtypeStruct((ROWS, 1), jnp.float32)),
  )(xw, degp)


def _tc_mid(accp, y1, dinv, b1, gamma, beta, W2):
  def body(a_ref, y_ref, di_ref, b1_ref, g_ref, be_ref, w2_ref,
           h_ref, y2_ref):
    dinv = di_ref[...]
    h = (a_ref[0] + a_ref[1] + y_ref[...]) * dinv + b1_ref[...]
    h_ref[...] = h
    hv = h[:N]
    mu = jnp.mean(hv, axis=0, keepdims=True)
    var = jnp.mean(jnp.square(hv - mu), axis=0, keepdims=True)
    hn = (h - mu) * lax.rsqrt(var + 1e-5) * g_ref[...] + be_ref[...]
    hr = jnp.maximum(hn, 0.0)
    hw = jnp.dot(hr, w2_ref[...], preferred_element_type=jnp.float32)
    y2_ref[...] = hw * dinv

  return pl.pallas_call(
      body,
      out_shape=(jax.ShapeDtypeStruct((ROWS, D), jnp.float32),
                 jax.ShapeDtypeStruct((ROWS, D), jnp.float32)),
  )(accp, y1, dinv, b1, gamma, beta, W2)


def _tc_final(accp, y2, dinv, b2):
  def body(a_ref, y_ref, di_ref, b2_ref, o_ref):
    o_ref[...] = (a_ref[0] + a_ref[1] + y_ref[...]) * di_ref[...] + b2_ref[...]

  return pl.pallas_call(
      body,
      out_shape=jax.ShapeDtypeStruct((ROWS, D), jnp.float32),
  )(accp, y2, dinv, b2)


@jax.jit
def kernel(x, edge_index, W1, b1, gamma, beta, W2, b2):
  E = edge_index.shape[1]
  totc = -(-E // CHUNK)
  totc = -(-totc // (NW * 8)) * (NW * 8)
  e_pad = totc * CHUNK

  src = edge_index[0]
  dst = edge_index[1]
  pad = jnp.full((e_pad - E,), N, jnp.int32)
  src_p = jnp.concatenate([src, pad]).reshape(totc, CHUNK)
  dst_p = jnp.concatenate([dst, pad]).reshape(totc, CHUNK)

  x_p = jnp.pad(x, ((0, ROWS - N), (0, 0)))
  b1r = b1.reshape(1, D)
  b2r = b2.reshape(1, D)
  gr = gamma.reshape(1, D)
  ber = beta.reshape(1, D)

  degp = _make_deg_kernel(totc)(dst_p)
  xw = _tc_matmul(x_p, W1)
  y1, dinv = _tc_scale(xw, degp)
  acc1 = _make_scatter_kernel(totc)(y1, src_p, dst_p)
  hidden_p, y2 = _tc_mid(acc1, y1, dinv, b1r, gr, ber, W2)
  acc2 = _make_scatter_kernel(totc)(y2, src_p, dst_p)
  out_p = _tc_final(acc2, y2, dinv, b2r)

  return out_p[:N], hidden_p[:N]


# speedup vs baseline: 1.5734x; 1.4379x over previous
import functools

import jax
import jax.numpy as jnp
from jax import lax
from jax.experimental import pallas as pl
from jax.experimental.pallas import tpu as pltpu
from jax.experimental.pallas import tpu_sc as plsc

N = 10000
D = 128
ROWS = 10240
NC = 2
NS = 16
NW = NC * NS
CHUNK = 128
RPT = ROWS // NS
ZB = 64
ZB2 = 32
DEGW = 128
G = 8
FAST_C = 0
FRAC = 0.65

_mesh = plsc.VectorSubcoreMesh(core_axis_name="c", subcore_axis_name="s")


def _split(totc):
  cpw_f = int(round(totc * FRAC / NS / G)) * G
  cpw_s = totc // NS - cpw_f
  assert cpw_s > 0 and cpw_s % G == 0
  return cpw_f, cpw_s


def _init_const_buf(ref, rows, width, value):
  per_row = width // 16

  def body(t, _):
    i = t // per_row
    k = t % per_row
    ref[i, pl.ds(k * 16, 16)] = jnp.full((16,), value, jnp.float32)
    return 0

  lax.fori_loop(0, rows * per_row, body, 0)


def _make_deg_kernel(totc):
  cpw = totc // NW

  @functools.partial(
      pl.kernel,
      out_type=jax.ShapeDtypeStruct((NC, ROWS, DEGW), jnp.float32),
      mesh=_mesh,
      scratch_types=[
          pltpu.VMEM((cpw, CHUNK), jnp.int32),
          pltpu.VMEM((CHUNK, DEGW), jnp.float32),
          pltpu.VMEM((ZB, DEGW), jnp.float32),
          pltpu.VMEM_SHARED((ROWS, DEGW), jnp.float32),
      ],
  )
  def deg_kernel(dst_hbm, out_hbm, idx_v, ones_v, zb_v, acc):
    c = lax.axis_index("c")
    s = lax.axis_index("s")
    wid = c * NS + s
    _init_const_buf(ones_v, CHUNK, DEGW, 1.0)
    _init_const_buf(zb_v, ZB, DEGW, 0.0)

    def zbody(t, _):
      pltpu.sync_copy(zb_v, acc.at[pl.ds(s * RPT + t * ZB, ZB)])
      return 0
    lax.fori_loop(0, RPT // ZB, zbody, 0)
    plsc.subcore_barrier()

    pltpu.sync_copy(dst_hbm.at[wid], idx_v)

    def body(j, _):
      pltpu.sync_copy(ones_v, acc.at[idx_v.at[j]], add=True)
      return 0
    lax.fori_loop(0, cpw, body, 0)
    plsc.subcore_barrier()

    pltpu.sync_copy(acc.at[pl.ds(s * RPT, RPT)],
                    out_hbm.at[c, pl.ds(s * RPT, RPT)])

  return deg_kernel


def _make_scatter_kernel(totc):
  cpw = totc // NW

  @functools.partial(
      pl.kernel,
      out_type=jax.ShapeDtypeStruct((NC, ROWS, D), jnp.float32),
      mesh=_mesh,
      scratch_types=[
          pltpu.VMEM((cpw, CHUNK), jnp.int32),
          pltpu.VMEM((cpw, CHUNK), jnp.int32),
          pltpu.VMEM((CHUNK, D), jnp.float32),
          pltpu.VMEM((ZB, D), jnp.float32),
          pltpu.VMEM_SHARED((ROWS, D), jnp.float32),
          pltpu.SemaphoreType.DMA,
      ],
  )
  def scat_kernel(y_hbm, src_hbm, dst_hbm, out_hbm, idx_s, idx_d, rows_v,
                  zb_v, acc, sem):
    c = lax.axis_index("c")
    s = lax.axis_index("s")
    wid = c * NS + s
    _init_const_buf(zb_v, ZB, D, 0.0)

    def zbody(t, _):
      pltpu.sync_copy(zb_v, acc.at[pl.ds(s * RPT + t * ZB, ZB)])
      return 0
    lax.fori_loop(0, RPT // ZB, zbody, 0)
    plsc.subcore_barrier()

    pltpu.sync_copy(src_hbm.at[wid], idx_s)
    pltpu.sync_copy(dst_hbm.at[wid], idx_d)

    def body(j, _):
      pltpu.async_copy(y_hbm.at[idx_s.at[j]], rows_v, sem).wait()
      pltpu.sync_copy(rows_v, acc.at[idx_d.at[j]], add=True)
      return 0

    lax.fori_loop(0, cpw, body, 0)
    plsc.subcore_barrier()

    pltpu.sync_copy(acc.at[pl.ds(s * RPT, RPT)],
                    out_hbm.at[c, pl.ds(s * RPT, RPT)])

  return scat_kernel


def _tc_matmul(x_p, W1):
  def body(x_ref, w_ref, o_ref):
    o_ref[...] = jnp.dot(x_ref[...], w_ref[...],
                         preferred_element_type=jnp.float32)

  return pl.pallas_call(
      body,
      out_shape=jax.ShapeDtypeStruct((ROWS, D), jnp.float32),
  )(x_p, W1)


def _tc_scale(xw, degp):
  def body(xw_ref, dp_ref, y_ref, dinv_ref):
    deg = dp_ref[0, :, 0:1] + dp_ref[1, :, 0:1] + 1.0
    dinv = lax.rsqrt(jnp.maximum(deg, 1e-12))
    y_ref[...] = xw_ref[...] * dinv
    dinv_ref[...] = dinv

  return pl.pallas_call(
      body,
      out_shape=(jax.ShapeDtypeStruct((ROWS, D), jnp.float32),
                 jax.ShapeDtypeStruct((ROWS, 1), jnp.float32)),
  )(xw, degp)


def _tc_mid(accp, y1, dinv, b1, gamma, beta, W2):
  def body(a_ref, y_ref, di_ref, b1_ref, g_ref, be_ref, w2_ref,
           h_ref, y2_ref):
    dinv = di_ref[...]
    h = (a_ref[0] + a_ref[1] + y_ref[...]) * dinv + b1_ref[...]
    h_ref[...] = h
    hv = h[:N]
    mu = jnp.mean(hv, axis=0, keepdims=True)
    var = jnp.mean(jnp.square(hv - mu), axis=0, keepdims=True)
    hn = (h - mu) * lax.rsqrt(var + 1e-5) * g_ref[...] + be_ref[...]
    hr = jnp.maximum(hn, 0.0)
    hw = jnp.dot(hr, w2_ref[...], preferred_element_type=jnp.float32)
    y2_ref[...] = hw * dinv

  return pl.pallas_call(
      body,
      out_shape=(jax.ShapeDtypeStruct((ROWS, D), jnp.float32),
                 jax.ShapeDtypeStruct((ROWS, D), jnp.float32)),
  )(accp, y1, dinv, b1, gamma, beta, W2)


def _tc_final(accp, y2, dinv, b2):
  def body(a_ref, y_ref, di_ref, b2_ref, o_ref):
    o_ref[...] = (a_ref[0] + a_ref[1] + y_ref[...]) * di_ref[...] + b2_ref[...]

  return pl.pallas_call(
      body,
      out_shape=jax.ShapeDtypeStruct((ROWS, D), jnp.float32),
  )(accp, y2, dinv, b2)


@jax.jit
def kernel(x, edge_index, W1, b1, gamma, beta, W2, b2):
  E = edge_index.shape[1]
  totc = -(-E // CHUNK)
  totc = -(-totc // NW) * NW
  e_pad = totc * CHUNK

  src = edge_index[0]
  dst = edge_index[1]
  pad = jnp.full((e_pad - E,), N, jnp.int32)
  src_p = jnp.concatenate([src, pad]).reshape(NW, totc // NW, CHUNK)
  dst_p = jnp.concatenate([dst, pad]).reshape(NW, totc // NW, CHUNK)

  x_p = jnp.pad(x, ((0, ROWS - N), (0, 0)))
  b1r = b1.reshape(1, D)
  b2r = b2.reshape(1, D)
  gr = gamma.reshape(1, D)
  ber = beta.reshape(1, D)

  degp = _make_deg_kernel(totc)(dst_p)
  xw = _tc_matmul(x_p, W1)
  y1, dinv = _tc_scale(xw, degp)
  acc1 = _make_scatter_kernel(totc)(y1, src_p, dst_p)
  hidden_p, y2 = _tc_mid(acc1, y1, dinv, b1r, gr, ber, W2)
  acc2 = _make_scatter_kernel(totc)(y2, src_p, dst_p)
  out_p = _tc_final(acc2, y2, dinv, b2r)

  return out_p[:N], hidden_p[:N]


# speedup vs baseline: 1.6359x; 1.0398x over previous
import functools

import jax
import jax.numpy as jnp
from jax import lax
from jax.experimental import pallas as pl
from jax.experimental.pallas import tpu as pltpu
from jax.experimental.pallas import tpu_sc as plsc

N = 10000
D = 128
ROWS = 10240
NC = 2
NS = 16
NW = NC * NS
CHUNK = 128
RPT = ROWS // NS
ZB = 64
ZB2 = 32
DEGW = 128
G = 8
FAST_C = 0
FRAC = 0.65

_mesh = plsc.VectorSubcoreMesh(core_axis_name="c", subcore_axis_name="s")


def _split(totc):
  cpw_f = int(round(totc * FRAC / NS / G)) * G
  cpw_s = totc // NS - cpw_f
  assert cpw_s > 0 and cpw_s % G == 0
  return cpw_f, cpw_s


def _init_const_buf(ref, rows, width, value):
  per_row = width // 16

  def body(t, _):
    i = t // per_row
    k = t % per_row
    ref[i, pl.ds(k * 16, 16)] = jnp.full((16,), value, jnp.float32)
    return 0

  lax.fori_loop(0, rows * per_row, body, 0)


def _make_deg_kernel(totc):
  cpw = totc // NW

  @functools.partial(
      pl.kernel,
      out_type=jax.ShapeDtypeStruct((NC, ROWS, DEGW), jnp.float32),
      mesh=_mesh,
      scratch_types=[
          pltpu.VMEM((cpw, CHUNK), jnp.int32),
          pltpu.VMEM((CHUNK, DEGW), jnp.float32),
          pltpu.VMEM((ZB, DEGW), jnp.float32),
          pltpu.VMEM_SHARED((ROWS, DEGW), jnp.float32),
      ],
  )
  def deg_kernel(dst_hbm, out_hbm, idx_v, ones_v, zb_v, acc):
    c = lax.axis_index("c")
    s = lax.axis_index("s")
    wid = c * NS + s
    _init_const_buf(ones_v, CHUNK, DEGW, 1.0)
    _init_const_buf(zb_v, ZB, DEGW, 0.0)

    def zbody(t, _):
      pltpu.sync_copy(zb_v, acc.at[pl.ds(s * RPT + t * ZB, ZB)])
      return 0
    lax.fori_loop(0, RPT // ZB, zbody, 0)
    plsc.subcore_barrier()

    pltpu.sync_copy(dst_hbm.at[wid], idx_v)

    def body(j, _):
      pltpu.sync_copy(ones_v, acc.at[idx_v.at[j]], add=True)
      return 0
    lax.fori_loop(0, cpw, body, 0)
    plsc.subcore_barrier()

    pltpu.sync_copy(acc.at[pl.ds(s * RPT, RPT)],
                    out_hbm.at[c, pl.ds(s * RPT, RPT)])

  return deg_kernel


def _make_scatter_kernel(totc):
  cpw = totc // NW

  @functools.partial(
      pl.kernel,
      out_type=jax.ShapeDtypeStruct((NC, ROWS, D), jnp.float32),
      mesh=_mesh,
      scratch_types=[
          pltpu.VMEM((cpw, CHUNK), jnp.int32),
          pltpu.VMEM((cpw, CHUNK), jnp.int32),
          pltpu.VMEM((CHUNK, D), jnp.float32),
          pltpu.VMEM((ZB, D), jnp.float32),
          pltpu.VMEM_SHARED((ROWS, D), jnp.float32),
          pltpu.SemaphoreType.DMA,
      ],
  )
  def scat_kernel(y_hbm, src_hbm, dst_hbm, out_hbm, idx_s, idx_d, rows_v,
                  zb_v, acc, sem):
    c = lax.axis_index("c")
    s = lax.axis_index("s")
    wid = c * NS + s
    _init_const_buf(zb_v, ZB, D, 0.0)

    def zbody(t, _):
      pltpu.sync_copy(zb_v, acc.at[pl.ds(s * RPT + t * ZB, ZB)])
      return 0
    lax.fori_loop(0, RPT // ZB, zbody, 0)
    plsc.subcore_barrier()

    pltpu.sync_copy(src_hbm.at[wid], idx_s)
    pltpu.sync_copy(dst_hbm.at[wid], idx_d)

    def body(j, _):
      pltpu.async_copy(y_hbm.at[idx_s.at[j]], rows_v, sem).wait()
      pltpu.sync_copy(rows_v, acc.at[idx_d.at[j]], add=True)
      return 0

    lax.fori_loop(0, cpw, body, 0)
    plsc.subcore_barrier()

    pltpu.sync_copy(acc.at[pl.ds(s * RPT, RPT)],
                    out_hbm.at[c, pl.ds(s * RPT, RPT)])

  return scat_kernel


def _tc_prep(x_p, W1, degp):
  def body(x_ref, w_ref, dp_ref, y_ref, dinv_ref):
    deg = dp_ref[0, :, 0:1] + dp_ref[1, :, 0:1] + 1.0
    dinv = lax.rsqrt(jnp.maximum(deg, 1e-12))
    xw = jnp.dot(x_ref[...], w_ref[...], preferred_element_type=jnp.float32)
    y_ref[...] = xw * dinv
    dinv_ref[...] = dinv

  return pl.pallas_call(
      body,
      out_shape=(jax.ShapeDtypeStruct((ROWS, D), jnp.float32),
                 jax.ShapeDtypeStruct((ROWS, 1), jnp.float32)),
  )(x_p, W1, degp)


def _tc_mid(accp, y1, dinv, b1, gamma, beta, W2):
  def body(a_ref, y_ref, di_ref, b1_ref, g_ref, be_ref, w2_ref,
           h_ref, y2_ref):
    dinv = di_ref[...]
    h = (a_ref[0] + a_ref[1] + y_ref[...]) * dinv + b1_ref[...]
    h_ref[...] = h
    hv = h[:N]
    mu = jnp.mean(hv, axis=0, keepdims=True)
    var = jnp.mean(jnp.square(hv - mu), axis=0, keepdims=True)
    hn = (h - mu) * lax.rsqrt(var + 1e-5) * g_ref[...] + be_ref[...]
    hr = jnp.maximum(hn, 0.0)
    hw = jnp.dot(hr, w2_ref[...], preferred_element_type=jnp.float32)
    y2_ref[...] = hw * dinv

  return pl.pallas_call(
      body,
      out_shape=(jax.ShapeDtypeStruct((ROWS, D), jnp.float32),
                 jax.ShapeDtypeStruct((ROWS, D), jnp.float32)),
  )(accp, y1, dinv, b1, gamma, beta, W2)


def _tc_final(accp, y2, dinv, b2):
  def body(a_ref, y_ref, di_ref, b2_ref, o_ref):
    o_ref[...] = (a_ref[0] + a_ref[1] + y_ref[...]) * di_ref[...] + b2_ref[...]

  return pl.pallas_call(
      body,
      out_shape=jax.ShapeDtypeStruct((ROWS, D), jnp.float32),
  )(accp, y2, dinv, b2)


@jax.jit
def kernel(x, edge_index, W1, b1, gamma, beta, W2, b2):
  E = edge_index.shape[1]
  totc = -(-E // CHUNK)
  totc = -(-totc // NW) * NW
  e_pad = totc * CHUNK

  src = edge_index[0]
  dst = edge_index[1]
  pad = jnp.full((e_pad - E,), N, jnp.int32)
  src_p = jnp.concatenate([src, pad]).reshape(NW, totc // NW, CHUNK)
  dst_p = jnp.concatenate([dst, pad]).reshape(NW, totc // NW, CHUNK)

  x_p = jnp.pad(x, ((0, ROWS - N), (0, 0)))
  b1r = b1.reshape(1, D)
  b2r = b2.reshape(1, D)
  gr = gamma.reshape(1, D)
  ber = beta.reshape(1, D)

  degp = _make_deg_kernel(totc)(dst_p)
  y1, dinv = _tc_prep(x_p, W1, degp)
  acc1 = _make_scatter_kernel(totc)(y1, src_p, dst_p)
  hidden_p, y2 = _tc_mid(acc1, y1, dinv, b1r, gr, ber, W2)
  acc2 = _make_scatter_kernel(totc)(y2, src_p, dst_p)
  out_p = _tc_final(acc2, y2, dinv, b2r)

  return out_p[:N], hidden_p[:N]
